# baseline (speedup 1.0000x reference)
import functools

import jax
import jax.numpy as jnp
from jax.experimental import pallas as pl
from jax.experimental.pallas import tpu as pltpu

N = 4096
D = 128
ALPHA = 2.0
BETA = 5.0
MARGIN = 0.5
EPSILON = 0.2

BM = 1024
NB = N // BM

_LOG2E = 1.4426950408889634


def _ms_loss_body(x_ref, trow_ref, tcol_ref, out_ref, acc_ref, sim_ref):
    i = pl.program_id(0)

    @pl.when(i == 0)
    def _init():
        acc_ref[0] = 0.0
        acc_ref[1] = 0.0

    x_blk = x_ref[pl.ds(i * BM, BM), :]
    sim_ref[...] = jax.lax.dot_general(
        x_blk, x_ref[...],
        dimension_numbers=(((1,), (1,)), ((), ())),
        preferred_element_type=jnp.float32,
    )
    sim = sim_ref[...]
    sim2 = sim_ref[...]

    t_rows = trow_ref[...]
    t_cols = tcol_ref[...]
    same = t_rows == t_cols

    inf = jnp.float32(jnp.inf)
    min_pos = jnp.min(
        jnp.where(same & (sim < 1.0), sim, inf), axis=1, keepdims=True)
    max_neg = jnp.max(jnp.where(same, -inf, sim), axis=1, keepdims=True)

    thr_p = jnp.minimum(jnp.float32(1.0), max_neg + EPSILON)
    thr_n = min_pos - EPSILON

    cp = jnp.float32(ALPHA * _LOG2E)
    cn = jnp.float32(BETA * _LOG2E)
    argp = sim2 * (-cp) + thr_p * cp
    argn = sim2 * cn + thr_n * (-cn)
    pos_scale = jnp.exp2(cp * (MARGIN - thr_p))
    neg_scale = jnp.exp2(cn * (thr_n - MARGIN))
    pos_sum = pos_scale * jnp.sum(
        jnp.where(same & (argp > 0.0), jnp.exp2(argp), 0.0),
        axis=1, keepdims=True)
    neg_sum = neg_scale * jnp.sum(
        jnp.where(same, 0.0, jnp.where(argn > 0.0, jnp.exp2(argn), 0.0)),
        axis=1, keepdims=True)

    valid = ((min_pos < inf) & (max_neg > -inf)
             & (pos_sum > 0.0) & (neg_sum > 0.0))
    row_loss = jnp.where(
        valid,
        jnp.log1p(pos_sum) * jnp.float32(1.0 / ALPHA)
        + jnp.log1p(neg_sum) * jnp.float32(1.0 / BETA),
        0.0)

    acc_ref[0] += jnp.sum(row_loss)
    acc_ref[1] += jnp.sum(valid.astype(jnp.float32))

    @pl.when(i == NB - 1)
    def _finalize():
        s = acc_ref[0]
        c = acc_ref[1]
        val = jnp.where(c > 0.0, s / jnp.maximum(c, 1.0), 0.0)
        out_ref[...] = jnp.broadcast_to(val, (1, 1))


@functools.partial(jax.jit, static_argnames=("interpret",))
def _ms_loss(x, t, interpret=False):
    trow = t.reshape(N, 1)
    tcol = t.reshape(1, N)
    loss = pl.pallas_call(
        _ms_loss_body,
        grid=(NB,),
        in_specs=[
            pl.BlockSpec((N, D), lambda i: (0, 0)),
            pl.BlockSpec((BM, 1), lambda i: (i, 0)),
            pl.BlockSpec((1, N), lambda i: (0, 0)),
        ],
        out_specs=pl.BlockSpec((1, 1), lambda i: (0, 0)),
        out_shape=jax.ShapeDtypeStruct((1, 1), jnp.float32),
        scratch_shapes=[pltpu.SMEM((2,), jnp.float32),
                        pltpu.VMEM((BM, N), jnp.float32)],
        compiler_params=pltpu.CompilerParams(
            dimension_semantics=("arbitrary",)),
        interpret=interpret,
    )(x, trow, tcol)
    return loss[0, 0]


def kernel(output, target):
    return _ms_loss(output, target)


# speedup vs baseline: 1.0859x; 1.0859x over previous
import functools

import jax
import jax.numpy as jnp
from jax.experimental import pallas as pl
from jax.experimental.pallas import tpu as pltpu

N = 4096
D = 128
ALPHA = 2.0
BETA = 5.0
MARGIN = 0.5
EPSILON = 0.2

BM = 2048
NB = N // BM
TC = 512
NT = N // TC

_LOG2E = 1.4426950408889634


def _tree(op, xs):
    while len(xs) > 1:
        xs = [op(xs[k], xs[k + 1]) for k in range(0, len(xs) - 1, 2)] + (
            [xs[-1]] if len(xs) % 2 else [])
    return xs[0]


def _ms_loss_body(x_ref, trow_ref, tcol_ref, out_ref, acc_ref, sim_ref):
    i = pl.program_id(0)

    @pl.when(i == 0)
    def _init():
        acc_ref[0] = 0.0
        acc_ref[1] = 0.0

    x_blk = x_ref[pl.ds(i * BM, BM), :]
    sim_ref[...] = jax.lax.dot_general(
        x_blk, x_ref[...],
        dimension_numbers=(((1,), (1,)), ((), ())),
        preferred_element_type=jnp.float32,
    )

    t_rows = trow_ref[...]
    inf = jnp.float32(jnp.inf)

    mins, maxs = [], []
    for t in range(NT):
        tile = sim_ref[:, t * TC:(t + 1) * TC]
        same = t_rows == tcol_ref[:, t * TC:(t + 1) * TC]
        mins.append(jnp.min(
            jnp.where(same & (tile < 1.0), tile, inf), axis=1, keepdims=True))
        maxs.append(jnp.max(
            jnp.where(same, -inf, tile), axis=1, keepdims=True))
    min_pos = _tree(jnp.minimum, mins)
    max_neg = _tree(jnp.maximum, maxs)

    thr_p = jnp.minimum(jnp.float32(1.0), max_neg + EPSILON)
    thr_n = min_pos - EPSILON

    ca = jnp.float32(-ALPHA * _LOG2E)
    cb = jnp.float32(ALPHA * MARGIN * _LOG2E)
    cc = jnp.float32(BETA * _LOG2E)
    cd = jnp.float32(-BETA * MARGIN * _LOG2E)
    psums, nsums = [], []
    for t in range(NT):
        tile = sim_ref[:, t * TC:(t + 1) * TC]
        same = t_rows == tcol_ref[:, t * TC:(t + 1) * TC]
        psums.append(jnp.sum(
            jnp.where(same & (tile < thr_p), jnp.exp2(tile * ca + cb), 0.0),
            axis=1, keepdims=True))
        nsums.append(jnp.sum(
            jnp.where(same, 0.0,
                      jnp.where(tile > thr_n, jnp.exp2(tile * cc + cd), 0.0)),
            axis=1, keepdims=True))
    pos_sum = _tree(jnp.add, psums)
    neg_sum = _tree(jnp.add, nsums)

    valid = ((min_pos < inf) & (max_neg > -inf)
             & (pos_sum > 0.0) & (neg_sum > 0.0))
    row_loss = jnp.where(
        valid,
        jnp.log1p(pos_sum) * jnp.float32(1.0 / ALPHA)
        + jnp.log1p(neg_sum) * jnp.float32(1.0 / BETA),
        0.0)

    acc_ref[0] += jnp.sum(row_loss)
    acc_ref[1] += jnp.sum(valid.astype(jnp.float32))

    @pl.when(i == NB - 1)
    def _finalize():
        s = acc_ref[0]
        c = acc_ref[1]
        val = jnp.where(c > 0.0, s / jnp.maximum(c, 1.0), 0.0)
        out_ref[...] = jnp.broadcast_to(val, (1, 1))


@functools.partial(jax.jit, static_argnames=("interpret",))
def _ms_loss(x, t, interpret=False):
    trow = t.reshape(N, 1)
    tcol = t.reshape(1, N)
    loss = pl.pallas_call(
        _ms_loss_body,
        grid=(NB,),
        in_specs=[
            pl.BlockSpec((N, D), lambda i: (0, 0)),
            pl.BlockSpec((BM, 1), lambda i: (i, 0)),
            pl.BlockSpec((1, N), lambda i: (0, 0)),
        ],
        out_specs=pl.BlockSpec((1, 1), lambda i: (0, 0)),
        out_shape=jax.ShapeDtypeStruct((1, 1), jnp.float32),
        scratch_shapes=[pltpu.SMEM((2,), jnp.float32),
                        pltpu.VMEM((BM, N), jnp.float32)],
        compiler_params=pltpu.CompilerParams(
            dimension_semantics=("arbitrary",)),
        interpret=interpret,
    )(x, trow, tcol)
    return loss[0, 0]


def kernel(output, target):
    return _ms_loss(output, target)
